# baseline (speedup 1.0000x reference)
import functools

import jax
import jax.numpy as jnp
from jax import lax
from jax.experimental import pallas as pl
from jax.experimental.pallas import tpu as pltpu
from jax.experimental.pallas import tpu_sc as plsc

NC = 2
NS = 16
NW = NC * NS
D = 128
DW = 144
CH = 128
N_OUT = 1024


def _make_agg(ndst_pad, ept):
  S = 4
  nr = ept // (S * CH)
  assert ept % (S * CH) == 0
  rows_z = ndst_pad // NS
  rows_o = N_OUT // NS
  mesh = plsc.VectorSubcoreMesh(
      core_axis_name="c", subcore_axis_name="s", num_cores=NC, num_subcores=NS)

  @functools.partial(
      pl.kernel,
      mesh=mesh,
      out_type=jax.ShapeDtypeStruct((NC, N_OUT, DW), jnp.float32),
      compiler_params=pltpu.CompilerParams(use_tc_tiling_on_sc=False),
      scratch_types=(
          pltpu.VMEM_SHARED((ndst_pad, DW), jnp.float32),
          [pltpu.VMEM((CH,), jnp.int32) for _ in range(S)],
          [pltpu.VMEM((CH,), jnp.int32) for _ in range(S)],
          [pltpu.VMEM((CH, DW), jnp.float32) for _ in range(S)],
          pltpu.SemaphoreType.DMA,
          pltpu.SemaphoreType.DMA,
          pltpu.SemaphoreType.DMA,
      ),
  )
  def agg(table_hbm, src_hbm, dst_hbm, zrow_hbm,
          acc_out, acc_sh, src_v, dst_v, rows_v, sem_i, sem_g, sem_s):
    c = lax.axis_index("c")
    s = lax.axis_index("s")
    wid = s * NC + c

    pltpu.sync_copy(zrow_hbm.at[pl.ds(0, rows_z)],
                    acc_sh.at[pl.ds(s * rows_z, rows_z)])
    plsc.subcore_barrier()

    base0 = wid * ept

    def body(r, carry):
      base = base0 + r * (S * CH)
      di = []
      for k in range(S):
        di.append(pltpu.async_copy(
            src_hbm.at[pl.ds(base + k * CH, CH)], src_v[k], sem_i))
        di.append(pltpu.async_copy(
            dst_hbm.at[pl.ds(base + k * CH, CH)], dst_v[k], sem_i))
      for d in di:
        d.wait()
      dg = [pltpu.async_copy(table_hbm.at[src_v[k]], rows_v[k], sem_g)
            for k in range(S)]
      for d in dg:
        d.wait()
      ds = [pltpu.async_copy(rows_v[k], acc_sh.at[dst_v[k]], sem_s, add=True)
            for k in range(S)]
      for d in ds:
        d.wait()
      return carry

    lax.fori_loop(0, nr, body, 0)
    plsc.subcore_barrier()

    pltpu.sync_copy(acc_sh.at[pl.ds(s * rows_o, rows_o)],
                    acc_out.at[c, pl.ds(s * rows_o, rows_o)])

  return agg


def _augment(table):
  n = table.shape[0]
  return jnp.concatenate(
      [table, jnp.ones((n, 1), jnp.float32), jnp.zeros((n, DW - D - 1), jnp.float32)],
      axis=1)


def _sage_update(acc, x_dst, W_l, b_l, W_r):
  def body(acc_ref, xt_ref, wl_ref, bl_ref, wr_ref, o_ref):
    ssum = acc_ref[0][:, :D] + acc_ref[1][:, :D]
    csum = acc_ref[0][:, D:D + 1] + acc_ref[1][:, D:D + 1]
    mean = ssum / jnp.maximum(csum, 1.0)
    t1 = lax.dot_general(mean, wl_ref[...], (((1,), (1,)), ((), ())),
                         preferred_element_type=jnp.float32)
    t2 = lax.dot_general(xt_ref[...], wr_ref[...], (((1,), (1,)), ((), ())),
                         preferred_element_type=jnp.float32)
    o_ref[...] = jnp.maximum(t1 + t2 + bl_ref[...], 0.0)

  return pl.pallas_call(
      body,
      out_shape=jax.ShapeDtypeStruct((N_OUT, D), jnp.float32),
  )(acc, x_dst, W_l, b_l.reshape(1, D), W_r)


def _pad_edges(edge_index, e_pad, dummy_dst):
  src = edge_index[0].astype(jnp.int32)
  dst = edge_index[1].astype(jnp.int32)
  n = e_pad - src.shape[0]
  src = jnp.concatenate([src, jnp.zeros((n,), jnp.int32)])
  dst = jnp.concatenate([dst, jnp.full((n,), dummy_dst, jnp.int32)])
  return src, dst


def kernel(x, edge_index1, edge_index2, n_target1, n_target2,
           W_l1, b_l1, W_r1, W_l2, b_l2, W_r2):
  ND1, EPT1 = 5120, 10240
  ND2, EPT2 = 1152, 2048

  src1, dst1 = _pad_edges(edge_index1, NW * EPT1, ND1 - 8)
  src2, dst2 = _pad_edges(edge_index2, NW * EPT2, ND2 - 8)

  zrow = jnp.zeros((ND1 // NS, DW), jnp.float32)

  agg1 = _make_agg(ND1, EPT1)
  acc1 = agg1(_augment(x), src1, dst1, zrow)
  h1 = _sage_update(acc1, x[:N_OUT], W_l1, b_l1, W_r1)

  agg2 = _make_agg(ND2, EPT2)
  acc2 = agg2(_augment(h1), src2, dst2, zrow)
  out = _sage_update(acc2, h1, W_l2, b_l2, W_r2)
  return out


# speedup vs baseline: 1.0654x; 1.0654x over previous
import functools

import jax
import jax.numpy as jnp
from jax import lax
from jax.experimental import pallas as pl
from jax.experimental.pallas import tpu as pltpu
from jax.experimental.pallas import tpu_sc as plsc

NC = 2
NS = 16
NW = NC * NS
D = 128
DW = 144
CH = 128
N_OUT = 1024


def _make_agg(ndst_pad, ept):
  S = 2
  nr = ept // (S * CH)
  assert ept % (S * CH) == 0
  rows_z = ndst_pad // NS
  rows_o = N_OUT // NS
  mesh = plsc.VectorSubcoreMesh(
      core_axis_name="c", subcore_axis_name="s", num_cores=NC, num_subcores=NS)

  @functools.partial(
      pl.kernel,
      mesh=mesh,
      out_type=jax.ShapeDtypeStruct((NC, N_OUT, DW), jnp.float32),
      compiler_params=pltpu.CompilerParams(use_tc_tiling_on_sc=False),
      scratch_types=(
          pltpu.VMEM_SHARED((ndst_pad, DW), jnp.float32),
          [pltpu.VMEM((CH,), jnp.int32) for _ in range(S)],
          [pltpu.VMEM((CH,), jnp.int32) for _ in range(S)],
          [pltpu.VMEM((CH, DW), jnp.float32) for _ in range(S)],
          pltpu.SemaphoreType.DMA,
          pltpu.SemaphoreType.DMA,
          pltpu.SemaphoreType.DMA,
      ),
  )
  def agg(table_hbm, src_hbm, dst_hbm, zrow_hbm,
          acc_out, acc_sh, src_v, dst_v, rows_v, sem_i, sem_g, sem_s):
    c = lax.axis_index("c")
    s = lax.axis_index("s")
    wid = s * NC + c

    pltpu.sync_copy(zrow_hbm.at[pl.ds(0, rows_z)],
                    acc_sh.at[pl.ds(s * rows_z, rows_z)])
    plsc.subcore_barrier()

    base0 = wid * ept

    def fetch_idx(k, off):
      pltpu.async_copy(src_hbm.at[pl.ds(off, CH)], src_v[k], sem_i)
      pltpu.async_copy(dst_hbm.at[pl.ds(off, CH)], dst_v[k], sem_i)

    def wait_idx(k):
      pltpu.make_async_copy(src_hbm.at[pl.ds(0, CH)], src_v[k], sem_i).wait()
      pltpu.make_async_copy(dst_hbm.at[pl.ds(0, CH)], dst_v[k], sem_i).wait()

    def fire_gather(k):
      pltpu.async_copy(table_hbm.at[src_v[k]], rows_v[k], sem_g)

    def wait_gather(k):
      pltpu.make_async_copy(table_hbm.at[src_v[k]], rows_v[k], sem_g).wait()

    def fire_scatter(k):
      pltpu.async_copy(rows_v[k], acc_sh.at[dst_v[k]], sem_s, add=True)

    def wait_scatter(k):
      pltpu.make_async_copy(rows_v[k], acc_sh.at[dst_v[k]], sem_s).wait()

    fetch_idx(0, base0)
    wait_idx(0)
    fire_gather(0)

    def body(r, carry):
      base = base0 + 2 * r * CH

      @pl.when(r > 0)
      def _():
        wait_scatter(1)
      fetch_idx(1, base + CH)
      wait_gather(0)
      fire_scatter(0)
      wait_idx(1)
      fire_gather(1)

      wait_scatter(0)
      @pl.when(r + 1 < nr)
      def _():
        fetch_idx(0, base + 2 * CH)
      wait_gather(1)
      fire_scatter(1)

      @pl.when(r + 1 < nr)
      def _():
        wait_idx(0)
        fire_gather(0)
      return carry

    lax.fori_loop(0, nr, body, 0)
    wait_scatter(1)
    plsc.subcore_barrier()

    pltpu.sync_copy(acc_sh.at[pl.ds(s * rows_o, rows_o)],
                    acc_out.at[c, pl.ds(s * rows_o, rows_o)])

  return agg


def _augment(table):
  n = table.shape[0]
  return jnp.concatenate(
      [table, jnp.ones((n, 1), jnp.float32), jnp.zeros((n, DW - D - 1), jnp.float32)],
      axis=1)


def _sage_update(acc, x_dst, W_l, b_l, W_r):
  def body(acc_ref, xt_ref, wl_ref, bl_ref, wr_ref, o_ref):
    ssum = acc_ref[0][:, :D] + acc_ref[1][:, :D]
    csum = acc_ref[0][:, D:D + 1] + acc_ref[1][:, D:D + 1]
    mean = ssum / jnp.maximum(csum, 1.0)
    t1 = lax.dot_general(mean, wl_ref[...], (((1,), (1,)), ((), ())),
                         preferred_element_type=jnp.float32)
    t2 = lax.dot_general(xt_ref[...], wr_ref[...], (((1,), (1,)), ((), ())),
                         preferred_element_type=jnp.float32)
    o_ref[...] = jnp.maximum(t1 + t2 + bl_ref[...], 0.0)

  return pl.pallas_call(
      body,
      out_shape=jax.ShapeDtypeStruct((N_OUT, D), jnp.float32),
  )(acc, x_dst, W_l, b_l.reshape(1, D), W_r)


def _pad_edges(edge_index, e_pad, dummy_dst):
  src = edge_index[0].astype(jnp.int32)
  dst = edge_index[1].astype(jnp.int32)
  n = e_pad - src.shape[0]
  src = jnp.concatenate([src, jnp.zeros((n,), jnp.int32)])
  dst = jnp.concatenate([dst, jnp.full((n,), dummy_dst, jnp.int32)])
  return src, dst


def kernel(x, edge_index1, edge_index2, n_target1, n_target2,
           W_l1, b_l1, W_r1, W_l2, b_l2, W_r2):
  ND1, EPT1 = 5120, 10240
  ND2, EPT2 = 1152, 2048

  src1, dst1 = _pad_edges(edge_index1, NW * EPT1, ND1 - 8)
  src2, dst2 = _pad_edges(edge_index2, NW * EPT2, ND2 - 8)

  zrow = jnp.zeros((ND1 // NS, DW), jnp.float32)

  agg1 = _make_agg(ND1, EPT1)
  acc1 = agg1(_augment(x), src1, dst1, zrow)
  h1 = _sage_update(acc1, x[:N_OUT], W_l1, b_l1, W_r1)

  agg2 = _make_agg(ND2, EPT2)
  acc2 = agg2(_augment(h1), src2, dst2, zrow)
  out = _sage_update(acc2, h1, W_l2, b_l2, W_r2)
  return out


# speedup vs baseline: 2.3558x; 2.2112x over previous
import functools

import jax
import jax.numpy as jnp
from jax import lax
from jax.experimental import pallas as pl
from jax.experimental.pallas import tpu as pltpu
from jax.experimental.pallas import tpu_sc as plsc

NC = 2
NS = 16
NW = NC * NS
D = 128
DW = 144
CH = 128
N_OUT = 1024


def _make_agg(ndst_pad, ept, filter_lt=None):
  nch_max = ept // CH
  assert ept % CH == 0
  rows_z = ndst_pad // NS
  rows_o = N_OUT // NS
  cap = ept + CH
  mesh = plsc.VectorSubcoreMesh(
      core_axis_name="c", subcore_axis_name="s", num_cores=NC, num_subcores=NS)

  scratch = [
      pltpu.VMEM_SHARED((ndst_pad, DW), jnp.float32),
      pltpu.VMEM((ept,), jnp.int32),
      pltpu.VMEM((ept,), jnp.int32),
      pltpu.VMEM((CH, DW), jnp.float32),
      pltpu.SemaphoreType.DMA,
  ]
  if filter_lt is not None:
    scratch += [pltpu.VMEM((cap,), jnp.int32), pltpu.VMEM((cap,), jnp.int32)]

  @functools.partial(
      pl.kernel,
      mesh=mesh,
      out_type=jax.ShapeDtypeStruct((NC, N_OUT, DW), jnp.float32),
      compiler_params=pltpu.CompilerParams(
          use_tc_tiling_on_sc=False, needs_layout_passes=False),
      scratch_types=tuple(scratch),
  )
  def agg(table_hbm, src_hbm, dst_hbm, zrow_hbm,
          acc_out, acc_sh, raw_src, raw_dst, rows_v, sem, *comp):
    c = lax.axis_index("c")
    s = lax.axis_index("s")
    wid = s * NC + c

    pltpu.sync_copy(zrow_hbm.at[pl.ds(0, rows_z)],
                    acc_sh.at[pl.ds(s * rows_z, rows_z)])

    base0 = wid * ept
    pltpu.sync_copy(src_hbm.at[pl.ds(base0, ept)], raw_src)
    pltpu.sync_copy(dst_hbm.at[pl.ds(base0, ept)], raw_dst)

    if filter_lt is not None:
      csrc, cdst = comp

      def comp_body(j, off):
        sv = raw_src[pl.ds(j * 16, 16)]
        dv = raw_dst[pl.ds(j * 16, 16)]
        mask = dv < filter_lt
        cum = plsc.cumsum(jnp.where(mask, 1, 0).astype(jnp.int32))
        pos = off + cum - 1
        plsc.store_scatter(csrc, [pos], sv, mask=mask)
        plsc.store_scatter(cdst, [pos], dv, mask=mask)
        return off + jnp.max(cum)

      n = lax.fori_loop(0, ept // 16, comp_body, 0)
      dummy_s = jnp.zeros((16,), jnp.int32)
      dummy_d = jnp.full((16,), ndst_pad - 8, jnp.int32)
      for t in range(CH // 16):
        csrc[pl.ds(n + t * 16, 16)] = dummy_s
        cdst[pl.ds(n + t * 16, 16)] = dummy_d
      nch = lax.shift_right_logical(n + (CH - 1), 7)
      src_idx, dst_idx = csrc, cdst
    else:
      nch = nch_max
      src_idx, dst_idx = raw_src, raw_dst

    def body(g, carry):
      b = g * CH
      pltpu.async_copy(table_hbm.at[src_idx.at[pl.ds(b, CH)]], rows_v,
                       sem).wait()
      pltpu.sync_copy(rows_v, acc_sh.at[dst_idx.at[pl.ds(b, CH)]], add=True)
      return carry

    lax.fori_loop(0, nch, body, 0)
    plsc.subcore_barrier()

    pltpu.sync_copy(acc_sh.at[pl.ds(s * rows_o, rows_o)],
                    acc_out.at[c, pl.ds(s * rows_o, rows_o)])

  return agg


def _augment(table):
  n = table.shape[0]
  return jnp.concatenate(
      [table, jnp.ones((n, 1), jnp.float32), jnp.zeros((n, DW - D - 1), jnp.float32)],
      axis=1)


def _sage_update(acc, x_dst, W_l, b_l, W_r):
  def body(acc_ref, xt_ref, wl_ref, bl_ref, wr_ref, o_ref):
    ssum = acc_ref[0][:, :D] + acc_ref[1][:, :D]
    csum = acc_ref[0][:, D:D + 1] + acc_ref[1][:, D:D + 1]
    mean = ssum / jnp.maximum(csum, 1.0)
    t1 = lax.dot_general(mean, wl_ref[...], (((1,), (1,)), ((), ())),
                         preferred_element_type=jnp.float32)
    t2 = lax.dot_general(xt_ref[...], wr_ref[...], (((1,), (1,)), ((), ())),
                         preferred_element_type=jnp.float32)
    o_ref[...] = jnp.maximum(t1 + t2 + bl_ref[...], 0.0)

  return pl.pallas_call(
      body,
      out_shape=jax.ShapeDtypeStruct((N_OUT, D), jnp.float32),
  )(acc, x_dst, W_l, b_l.reshape(1, D), W_r)


def _pad_edges(edge_index, e_pad, dummy_dst):
  src = edge_index[0].astype(jnp.int32)
  dst = edge_index[1].astype(jnp.int32)
  n = e_pad - src.shape[0]
  src = jnp.concatenate([src, jnp.zeros((n,), jnp.int32)])
  dst = jnp.concatenate([dst, jnp.full((n,), dummy_dst, jnp.int32)])
  return src, dst


def kernel(x, edge_index1, edge_index2, n_target1, n_target2,
           W_l1, b_l1, W_r1, W_l2, b_l2, W_r2):
  ND1, EPT1 = 5120, 10240
  ND2, EPT2 = 1152, 2048

  src1, dst1 = _pad_edges(edge_index1, NW * EPT1, ND1 - 8)
  src2, dst2 = _pad_edges(edge_index2, NW * EPT2, ND2 - 8)

  zrow = jnp.zeros((ND1 // NS, DW), jnp.float32)

  agg1 = _make_agg(ND1, EPT1, filter_lt=N_OUT)
  acc1 = agg1(_augment(x), src1, dst1, zrow)
  h1 = _sage_update(acc1, x[:N_OUT], W_l1, b_l1, W_r1)

  agg2 = _make_agg(ND2, EPT2)
  acc2 = agg2(_augment(h1), src2, dst2, zrow)
  out = _sage_update(acc2, h1, W_l2, b_l2, W_r2)
  return out
